# baseline (speedup 1.0000x reference)
import functools

import jax
import jax.numpy as jnp
from jax import lax
from jax.experimental import pallas as pl
from jax.experimental.pallas import tpu as pltpu
from jax.experimental.pallas import tpu_sc as plsc

N = 2048
D = 768
E = 8
K = 2

_NC = 2
_NS = 16
_NW = _NC * _NS
_LN = 16
_TPW = N // _NW
_GPW = _TPW // _LN


def _gate_kernel(x_ref, wg_ref, bg_ref, zt_ref):
    zt = lax.dot_general(
        wg_ref[...], x_ref[...], (((1,), (1,)), ((), ())),
        preferred_element_type=jnp.float32,
    )
    zt_ref[0] = zt + bg_ref[...]


def _sc_counts_body(zt_hbm, out_hbm, zv, accs):
    wid = lax.axis_index("s") * _NC + lax.axis_index("c")
    pltpu.sync_copy(zt_hbm.at[wid], zv)
    acc = [jnp.zeros((_LN,), jnp.float32) for _ in range(E)]
    one = jnp.ones((_LN,), jnp.float32)
    zero = jnp.zeros((_LN,), jnp.float32)
    for j in range(_GPW):
        zs = [zv[e, pl.ds(j * _LN, _LN)] for e in range(E)]
        m1 = zs[0]
        for e in range(1, E):
            m1 = jnp.maximum(m1, zs[e])
        i1 = jnp.full((_LN,), E, jnp.int32)
        for e in reversed(range(E)):
            i1 = jnp.where(zs[e] == m1, e, i1)
        neg = jnp.full((_LN,), -3.0e38, jnp.float32)
        m2 = neg
        for e in range(E):
            m2 = jnp.maximum(m2, jnp.where(i1 == e, neg, zs[e]))
        i2 = jnp.full((_LN,), E, jnp.int32)
        for e in reversed(range(E)):
            i2 = jnp.where((zs[e] == m2) & (i1 != e), e, i2)
        for e in range(E):
            acc[e] = acc[e] + jnp.where(i1 == e, one, zero)
            acc[e] = acc[e] + jnp.where(i2 == e, one, zero)
    for e in range(E):
        accs[e, :] = acc[e]
    pltpu.sync_copy(accs, out_hbm.at[wid])


def _sc_counts(zt):
    mesh = plsc.VectorSubcoreMesh(core_axis_name="c", subcore_axis_name="s")
    fn = pl.kernel(
        _sc_counts_body,
        mesh=mesh,
        out_type=jax.ShapeDtypeStruct((_NW, E, _LN), jnp.float32),
        scratch_types=[
            pltpu.VMEM((E, _TPW), jnp.float32),
            pltpu.VMEM((E, _LN), jnp.float32),
        ],
    )
    return fn(zt)


def _expert_acc_kernel(cp_ref, x_ref, we_ref, be_ref, out_ref):
    e = pl.program_id(0)
    w = we_ref[0]
    z = lax.dot_general(
        x_ref[...], w, (((1,), (1,)), ((), ())),
        preferred_element_type=jnp.float32,
    )
    r = jnp.maximum(z + be_ref[0], 0.0)
    cp = cp_ref[...]
    row = jax.lax.broadcasted_iota(jnp.int32, cp.shape, 0)
    sel = (row % E) == e
    s = jnp.sum(jnp.where(sel, cp, 0.0), axis=(0, 1), keepdims=True)
    contrib = r * (s * (1.0 / float(N * K)))

    @pl.when(e == 0)
    def _():
        out_ref[...] = contrib

    @pl.when(e > 0)
    def _():
        out_ref[...] += contrib


def kernel(x, Wg, bg, We, be):
    zt = pl.pallas_call(
        _gate_kernel,
        grid=(_NW,),
        in_specs=[
            pl.BlockSpec((_TPW, D), lambda w: (w, 0)),
            pl.BlockSpec((E, D), lambda w: (0, 0)),
            pl.BlockSpec((E, 1), lambda w: (0, 0)),
        ],
        out_specs=pl.BlockSpec((1, E, _TPW), lambda w: (w, 0, 0)),
        out_shape=jax.ShapeDtypeStruct((_NW, E, _TPW), jnp.float32),
    )(x, Wg, bg.reshape(E, 1))

    counts_partial = _sc_counts(zt).reshape(_NW * E, _LN)

    out = pl.pallas_call(
        _expert_acc_kernel,
        grid=(E,),
        in_specs=[
            pl.BlockSpec((_NW * E, _LN), lambda e: (0, 0)),
            pl.BlockSpec((N, D), lambda e: (0, 0)),
            pl.BlockSpec((1, D, D), lambda e: (e, 0, 0)),
            pl.BlockSpec((1, 1, D), lambda e: (e, 0, 0)),
        ],
        out_specs=pl.BlockSpec((N, D), lambda e: (0, 0)),
        out_shape=jax.ShapeDtypeStruct((N, D), jnp.float32),
    )(counts_partial, x, We, be.reshape(E, 1, D))
    return out


# speedup vs baseline: 1.2018x; 1.2018x over previous
import functools

import jax
import jax.numpy as jnp
from jax import lax
from jax.experimental import pallas as pl
from jax.experimental.pallas import tpu as pltpu
from jax.experimental.pallas import tpu_sc as plsc

N = 2048
D = 768
E = 8
K = 2

_NC = 2
_NS = 16
_NW = _NC * _NS
_LN = 16
_TPW = N // _NW
_GPW = _TPW // _LN


def _gate_kernel(x_ref, wg_ref, bg_ref, zt_ref):
    zt = lax.dot_general(
        wg_ref[...], x_ref[...], (((1,), (1,)), ((), ())),
        preferred_element_type=jnp.float32,
    )
    zt_ref[...] = zt + bg_ref[...]


def _sc_counts_body(zt_hbm, out_hbm, zv, accs):
    wid = lax.axis_index("s") * _NC + lax.axis_index("c")
    base = wid * _TPW
    for e in range(E):
        pltpu.sync_copy(zt_hbm.at[e, pl.ds(base, _TPW)], zv.at[e])
    acc = [jnp.zeros((_LN,), jnp.float32) for _ in range(E)]
    one = jnp.ones((_LN,), jnp.float32)
    zero = jnp.zeros((_LN,), jnp.float32)
    for j in range(_GPW):
        zs = [zv[e, pl.ds(j * _LN, _LN)] for e in range(E)]
        m1 = zs[0]
        for e in range(1, E):
            m1 = jnp.maximum(m1, zs[e])
        i1 = jnp.full((_LN,), E, jnp.int32)
        for e in reversed(range(E)):
            i1 = jnp.where(zs[e] == m1, e, i1)
        neg = jnp.full((_LN,), -3.0e38, jnp.float32)
        m2 = neg
        for e in range(E):
            m2 = jnp.maximum(m2, jnp.where(i1 == e, neg, zs[e]))
        i2 = jnp.full((_LN,), E, jnp.int32)
        for e in reversed(range(E)):
            i2 = jnp.where((zs[e] == m2) & (i1 != e), e, i2)
        for e in range(E):
            acc[e] = acc[e] + jnp.where(i1 == e, one, zero)
            acc[e] = acc[e] + jnp.where(i2 == e, one, zero)
    for e in range(E):
        accs[e, :] = acc[e]
    pltpu.sync_copy(accs, out_hbm.at[wid])


def _sc_counts(zt):
    mesh = plsc.VectorSubcoreMesh(core_axis_name="c", subcore_axis_name="s")
    fn = pl.kernel(
        _sc_counts_body,
        mesh=mesh,
        out_type=jax.ShapeDtypeStruct((_NW, E, _LN), jnp.float32),
        scratch_types=[
            pltpu.VMEM((E, _TPW), jnp.float32),
            pltpu.VMEM((E, _LN), jnp.float32),
        ],
    )
    return fn(zt)


def _expert_acc_kernel(cp_ref, x_ref, we_ref, be_ref, out_ref):
    e = pl.program_id(0)
    w = we_ref[0]
    z = lax.dot_general(
        x_ref[...], w, (((1,), (1,)), ((), ())),
        preferred_element_type=jnp.float32,
    )
    r = jnp.maximum(z + be_ref[0], 0.0)
    cp = cp_ref[...]
    row = jax.lax.broadcasted_iota(jnp.int32, cp.shape, 0)
    sel = (row % E) == e
    s = jnp.sum(jnp.where(sel, cp, 0.0), axis=(0, 1), keepdims=True)
    contrib = r * (s * (1.0 / float(N * K)))

    @pl.when(e == 0)
    def _():
        out_ref[...] = contrib

    @pl.when(e > 0)
    def _():
        out_ref[...] += contrib


def kernel(x, Wg, bg, We, be):
    zt = pl.pallas_call(
        _gate_kernel,
        out_shape=jax.ShapeDtypeStruct((E, N), jnp.float32),
    )(x, Wg, bg.reshape(E, 1))

    counts_partial = _sc_counts(zt).reshape(_NW * E, _LN)

    out = pl.pallas_call(
        _expert_acc_kernel,
        grid=(E,),
        in_specs=[
            pl.BlockSpec((_NW * E, _LN), lambda e: (0, 0)),
            pl.BlockSpec((N, D), lambda e: (0, 0)),
            pl.BlockSpec((1, D, D), lambda e: (e, 0, 0)),
            pl.BlockSpec((1, 1, D), lambda e: (e, 0, 0)),
        ],
        out_specs=pl.BlockSpec((N, D), lambda e: (0, 0)),
        out_shape=jax.ShapeDtypeStruct((N, D), jnp.float32),
    )(counts_partial, x, We, be.reshape(E, 1, D))
    return out


# speedup vs baseline: 1.9070x; 1.5868x over previous
import jax
import jax.numpy as jnp
from jax import lax
from jax.experimental import pallas as pl
from jax.experimental.pallas import tpu as pltpu

N = 2048
D = 768
E = 8
K = 2


def _moe_kernel(x_ref, wg_ref, bg_ref, we_ref, be_ref, out_ref, scale_ref):
    e = pl.program_id(0)

    @pl.when(e == 0)
    def _():
        logits = lax.dot_general(
            x_ref[...], wg_ref[...], (((1,), (1,)), ((), ())),
            preferred_element_type=jnp.float32,
        ) + bg_ref[...]
        idx = lax.broadcasted_iota(jnp.int32, logits.shape, 1)
        m1 = jnp.max(logits, axis=1, keepdims=True)
        i1 = jnp.min(jnp.where(logits == m1, idx, E), axis=1, keepdims=True)
        oh1 = idx == i1
        masked = jnp.where(oh1, -jnp.inf, logits)
        m2 = jnp.max(masked, axis=1, keepdims=True)
        i2 = jnp.min(jnp.where(masked == m2, idx, E), axis=1, keepdims=True)
        oh2 = idx == i2
        cnt = jnp.sum(oh1.astype(jnp.float32) + oh2.astype(jnp.float32), axis=0)
        scale_ref[...] = (cnt / float(N * K)).reshape(1, E)

    w = we_ref[0]
    z = lax.dot_general(
        x_ref[...], w, (((1,), (1,)), ((), ())),
        preferred_element_type=jnp.float32,
    )
    r = jnp.maximum(z + be_ref[0], 0.0)
    sel = lax.broadcasted_iota(jnp.int32, (1, E), 1) == e
    s = jnp.sum(jnp.where(sel, scale_ref[...], 0.0), axis=(0, 1), keepdims=True)
    contrib = r * s

    @pl.when(e == 0)
    def _():
        out_ref[...] = contrib

    @pl.when(e > 0)
    def _():
        out_ref[...] += contrib


def kernel(x, Wg, bg, We, be):
    out = pl.pallas_call(
        _moe_kernel,
        grid=(E,),
        in_specs=[
            pl.BlockSpec((N, D), lambda e: (0, 0)),
            pl.BlockSpec((E, D), lambda e: (0, 0)),
            pl.BlockSpec((1, E), lambda e: (0, 0)),
            pl.BlockSpec((1, D, D), lambda e: (e, 0, 0)),
            pl.BlockSpec((1, 1, D), lambda e: (e, 0, 0)),
        ],
        out_specs=pl.BlockSpec((N, D), lambda e: (0, 0)),
        out_shape=jax.ShapeDtypeStruct((N, D), jnp.float32),
        scratch_shapes=[pltpu.VMEM((1, E), jnp.float32)],
    )(x, Wg, bg.reshape(1, E), We, be.reshape(E, 1, D))
    return out
